# baseline (speedup 1.0000x reference)
import functools

import jax
import jax.numpy as jnp
import numpy as np
from jax import lax
from jax.experimental import pallas as pl
from jax.experimental.pallas import tpu as pltpu
from jax.experimental.pallas import tpu_sc as plsc

_ANCHORS = np.array(
    [10, 13, 16, 30, 33, 23, 30, 61, 62, 45, 59, 119, 116, 90, 156, 198, 373, 326],
    dtype=np.float32,
).reshape(3, 3, 2)
_BAL = (4.0, 1.0, 0.4)
_OFF = ((0.0, 0.0), (1.0, 0.0), (0.0, 1.0), (-1.0, 0.0), (0.0, -1.0))
_GRIDS = ((80, 80), (40, 40), (20, 20))
_B = 16
_N = 256
_NCELL = 5 * _N
_EPS = 1e-7


def _softplus0(x):
    return jnp.maximum(x, 0.0) + jnp.log1p(jnp.exp(-jnp.abs(x)))


_ATAN_C = (9.9999999755e-01, -3.3333282296e-01, 1.9998230640e-01,
           -1.4261573680e-01, 1.0940198965e-01, -8.3720639484e-02,
           5.7463557856e-02, -3.0717508912e-02, 1.0680719451e-02,
           -1.7437011450e-03)


def _atan_pos(z):
    big = z > 1.0
    t = jnp.where(big, 1.0 / z, z)
    u = t * t
    p = jnp.full_like(u, _ATAN_C[-1])
    for c in _ATAN_C[-2::-1]:
        p = p * u + c
    at = t * p
    return jnp.where(big, (np.pi / 2) - at, at)


def _make_sc_body(lvl):
    H, W = _GRIDS[lvl]

    def _sc_body(pt, nb_hbm, cx_hbm, cy_hbm, img_hbm, cells,
                 nbv, cxv, cyv, imgv, idxv, slab, sem):
        nc = 2
        w = lax.axis_index("s") * nc + lax.axis_index("c")
        pltpu.sync_copy(nb_hbm, nbv.at[pl.ds(0, _N)])
        pltpu.sync_copy(nb_hbm, nbv.at[pl.ds(_N, _N)])
        pltpu.sync_copy(cx_hbm, cxv.at[pl.ds(0, _N)])
        pltpu.sync_copy(cx_hbm, cxv.at[pl.ds(_N, _N)])
        pltpu.sync_copy(cy_hbm, cyv.at[pl.ds(0, _N)])
        pltpu.sync_copy(cy_hbm, cyv.at[pl.ds(_N, _N)])
        pltpu.sync_copy(img_hbm, imgv)
        imgw = imgv[pl.ds(0, 16)]
        imgh = imgv[pl.ds(16, 16)]
        lane = lax.iota(jnp.int32, 16)
        base = w * 40
        gx_scale = 1.0 / (imgw / float(W))
        gy_scale = 1.0 / (imgh / float(H))
        for off in (0, 16, 32):
            ids = jnp.minimum(base + off + lane, _NCELL - 1)
            o = lax.shift_right_logical(ids, 8)
            xo = (jnp.where(o == 1, 1.0, 0.0)
                  + jnp.where(o == 3, -1.0, 0.0))
            yo = (jnp.where(o == 2, 1.0, 0.0)
                  + jnp.where(o == 4, -1.0, 0.0))
            t0 = (base + off) % _N
            cxt = cxv[pl.ds(t0, 16)] * gx_scale - xo
            cyt = cyv[pl.ds(t0, 16)] * gy_scale - yo
            nbt = nbv[pl.ds(t0, 16)]
            gi = jnp.clip(cxt.astype(jnp.int32), 0, W - 1)
            gj = jnp.clip(cyt.astype(jnp.int32), 0, H - 1)
            b = jnp.clip(nbt.astype(jnp.int32), 0, _B - 1)
            idxv[pl.ds(off, 16)] = b
            idxv[pl.ds(48 + off, 16)] = gj
            idxv[pl.ds(96 + off, 16)] = gi

        gb = [idxv[pl.ds(0, 16)], idxv[pl.ds(16, 16)], idxv[pl.ds(32, 16)]]
        gjv = [idxv[pl.ds(48, 16)], idxv[pl.ds(64, 16)], idxv[pl.ds(80, 16)]]
        giv = [idxv[pl.ds(96, 16)], idxv[pl.ds(112, 16)],
               idxv[pl.ds(128, 16)]]
        for k in range(40):
            q, r = divmod(k, 16)
            b = gb[q][r]
            gj = gjv[q][r]
            gi = giv[q][r]
            if lvl == 2:
                ba = pl.multiple_of((b // 8) * 8, 8)
                src = pt.at[pl.ds(gj, 1), pl.ds(gi, 1), pl.ds(ba, 8)]
            else:
                gia = pl.multiple_of((gi // 8) * 8, 8)
                src = pt.at[pl.ds(b, 1), pl.ds(gj, 1), pl.ds(gia, 8)]
            pltpu.async_copy(src, slab.at[pl.ds(k, 1), pl.ds(0, 1)], sem)
        pltpu.make_async_copy(cells.at[pl.ds(0, 40)], slab, sem).wait()
        pltpu.sync_copy(slab, cells.at[pl.ds(base, 40)])

    return _sc_body


@functools.cache
def _sc_gather_fn(lvl):
    mesh = plsc.VectorSubcoreMesh(core_axis_name="c", subcore_axis_name="s")
    return pl.kernel(
        _make_sc_body(lvl),
        mesh=mesh,
        out_type=jax.ShapeDtypeStruct((_NCELL, 1, 8, 255), jnp.float32),
        scratch_types=[
            pltpu.VMEM((512,), jnp.float32),
            pltpu.VMEM((512,), jnp.float32),
            pltpu.VMEM((512,), jnp.float32),
            pltpu.VMEM((32,), jnp.float32),
            pltpu.VMEM((144,), jnp.int32),
            pltpu.VMEM((40, 1, 8, 255), jnp.float32),
            pltpu.SemaphoreType.DMA,
        ],
    )


def _obj_body(p0_ref, p1_ref, p2_ref, out_ref, acc_ref):
    g = pl.program_id(0)

    @pl.when(g == 0)
    def _init():
        acc_ref[0] = 0.0
        acc_ref[1] = 0.0
        acc_ref[2] = 0.0

    @pl.when(g < 160)
    def _l01():
        s0 = 0.0
        s1 = 0.0
        for a in range(3):
            c = 85 * a + 4
            s0 += jnp.sum(_softplus0(p0_ref[0, :, :, c]))
            s1 += jnp.sum(_softplus0(p1_ref[0, :, :, c]))
        acc_ref[0] += s0
        acc_ref[1] += s1

    @pl.when(g >= 160)
    def _l2():
        s2 = 0.0
        for a in range(3):
            c = 85 * a + 4
            s2 += jnp.sum(_softplus0(p2_ref[:, :, :, c]))
        acc_ref[2] += s2

    @pl.when(g == pl.num_programs(0) - 1)
    def _final():
        out_ref[0] = acc_ref[0]
        out_ref[1] = acc_ref[1]
        out_ref[2] = acc_ref[2]
        out_ref[3] = 0.0


def _obj_main(pt0, pt1, pt2):
    def m0(g):
        gc = jnp.minimum(g, 159)
        return (gc // 10, gc % 10, 0, 0)

    def m2(g):
        return (jnp.maximum(g, 160) - 160, 0, 0, 0)

    return pl.pallas_call(
        _obj_body,
        grid=(170,),
        in_specs=[
            pl.BlockSpec((1, 8, 80, 255), m0),
            pl.BlockSpec((1, 4, 40, 255), m0),
            pl.BlockSpec((2, 20, _B, 255), m2),
        ],
        out_specs=pl.BlockSpec(memory_space=pltpu.SMEM),
        out_shape=jax.ShapeDtypeStruct((4,), jnp.float32),
        scratch_shapes=[pltpu.SMEM((4,), jnp.float32)],
    )(pt0, pt1, pt2)


def _loss_body(c0_ref, c1_ref, c2_ref, tt_ref, img_ref, objs_ref, out_ref,
               acc_ref):
    g = pl.program_id(0)
    imgw = img_ref[0, 0]
    imgh = img_ref[0, 1]
    cls_ = tt_ref[:, 1:2]
    xo = jnp.where(g == 1, 1.0, 0.0) + jnp.where(g == 3, -1.0, 0.0)
    yo = jnp.where(g == 2, 1.0, 0.0) + jnp.where(g == 4, -1.0, 0.0)

    @pl.when(g == 0)
    def _init():
        for i in range(12):
            acc_ref[3 + i] = 0.0

    for lvl, (H, W) in enumerate(_GRIDS):
        c_ref = (c0_ref, c1_ref, c2_ref)[lvl]
        sx = imgw / W
        sy = imgh / H
        gxs = 1.0 / sx
        gys = 1.0 / sy
        cx = tt_ref[:, 2:3] * gxs
        cy = tt_ref[:, 3:4] * gys
        gw = tt_ref[:, 4:5] * gxs
        gh = tt_ref[:, 5:6] * gys
        cidx = jnp.clip(cls_.astype(jnp.int32) - 1, 0, 79)
        oneh = (lax.broadcasted_iota(jnp.int32, (_N, 80), 1)
                == cidx).astype(jnp.float32)
        gxf = cx - xo
        gyf = cy - yo
        j1 = (gxf >= 0) & (gxf < W) & (gyf >= 0) & (gyf < H)
        gxi = gxf.astype(jnp.int32).astype(jnp.float32)
        gyi = gyf.astype(jnp.int32).astype(jnp.float32)
        if lvl == 2:
            bi = jnp.clip(tt_ref[:, 0:1].astype(jnp.int32), 0, _B - 1)
            rt = bi - 8 * (bi // 8)
        else:
            gii = jnp.clip(gxf.astype(jnp.int32), 0, W - 1)
            rt = gii - 8 * (gii // 8)
        psc255 = c_ref[:, 0, :] * (rt == 0).astype(jnp.float32)
        for r in range(1, 8):
            psc255 += c_ref[:, r, :] * (rt == r).astype(jnp.float32)
        box_s = 0.0
        corr_s = 0.0
        cls_s = 0.0
        nv_s = 0.0
        for a in range(3):
            aw = _ANCHORS[lvl, a, 0] / sx
            ah = _ANCHORS[lvl, a, 1] / sy
            rw = gw / aw
            rh = gh / ah
            j2 = (jnp.maximum(jnp.maximum(rw, 1.0 / rw),
                              jnp.maximum(rh, 1.0 / rh)) < 4.0)
            mf = jnp.where(j1 & j2, 1.0, 0.0)
            psc = psc255[:, 85 * a:85 * a + 85]
            px = 3.0 * jax.nn.sigmoid(psc[:, 0:1]) - 1.0
            py = 3.0 * jax.nn.sigmoid(psc[:, 1:2]) - 1.0
            sw = jax.nn.sigmoid(psc[:, 2:3])
            sh = jax.nn.sigmoid(psc[:, 3:4])
            pw = 4.0 * sw * sw * aw
            ph = 4.0 * sh * sh * ah
            tbx = cx - gxi
            tby = cy - gyi
            b1x1 = px - pw * 0.5
            b1x2 = px + pw * 0.5
            b1y1 = py - ph * 0.5
            b1y2 = py + ph * 0.5
            b2x1 = tbx - gw * 0.5
            b2x2 = tbx + gw * 0.5
            b2y1 = tby - gh * 0.5
            b2y2 = tby + gh * 0.5
            inter = (jnp.clip(jnp.minimum(b1x2, b2x2)
                              - jnp.maximum(b1x1, b2x1), 0.0)
                     * jnp.clip(jnp.minimum(b1y2, b2y2)
                                - jnp.maximum(b1y1, b2y1), 0.0))
            union = pw * ph + gw * gh - inter + _EPS
            iou = inter / union
            cw = jnp.maximum(b1x2, b2x2) - jnp.minimum(b1x1, b2x1)
            chh = jnp.maximum(b1y2, b2y2) - jnp.minimum(b1y1, b2y1)
            c2 = cw * cw + chh * chh + _EPS
            rho2 = ((b2x1 + b2x2 - b1x1 - b1x2) ** 2
                    + (b2y1 + b2y2 - b1y1 - b1y2) ** 2) * 0.25
            v = ((4.0 / np.pi ** 2)
                 * (_atan_pos(gw / (gh + _EPS))
                    - _atan_pos(pw / (ph + _EPS))) ** 2)
            alpha = v / (v - iou + (1.0 + _EPS))
            ciou = iou - (rho2 / c2 + v * alpha)
            box_s += jnp.sum((1.0 - ciou) * mf)
            corr_s += jnp.sum(psc[:, 4:5] * jnp.clip(ciou, 0.0) * mf)
            nv_s += jnp.sum(mf)
            xl = psc[:, 5:85]
            bce = (jnp.maximum(xl, 0.0) - xl * oneh
                   + jnp.log1p(jnp.exp(-jnp.abs(xl))))
            cls_s += jnp.sum(bce * mf)
        acc_ref[3 + lvl] += box_s
        acc_ref[6 + lvl] += corr_s
        acc_ref[9 + lvl] += cls_s
        acc_ref[12 + lvl] += nv_s

    @pl.when(g == pl.num_programs(0) - 1)
    def _final():
        lbox = 0.0
        lobj = 0.0
        lcls = 0.0
        for lvl, (H, W) in enumerate(_GRIDS):
            denom = jnp.maximum(acc_ref[12 + lvl], 1.0)
            lbox += acc_ref[3 + lvl] / denom
            lcls += acc_ref[9 + lvl] / (denom * 80.0)
            lobj += ((objs_ref[lvl] - acc_ref[6 + lvl])
                     / (_B * 3 * H * W)) * _BAL[lvl]
        lbox = lbox * 3.54
        lobj = lobj * 64.3
        lcls = lcls * 37.4
        loss = lbox + lobj + lcls
        out_ref[0] = loss
        out_ref[1] = lbox
        out_ref[2] = lobj
        out_ref[3] = lcls


def _loss_main(cells0, cells1, cells2, tt, img, objs):
    return pl.pallas_call(
        _loss_body,
        grid=(5,),
        in_specs=[
            pl.BlockSpec((_N, 8, 255), lambda g: (g, 0, 0)),
            pl.BlockSpec((_N, 8, 255), lambda g: (g, 0, 0)),
            pl.BlockSpec((_N, 8, 255), lambda g: (g, 0, 0)),
            pl.BlockSpec((_N, 6), lambda g: (0, 0)),
            pl.BlockSpec(memory_space=pltpu.SMEM),
            pl.BlockSpec(memory_space=pltpu.SMEM),
        ],
        out_specs=pl.BlockSpec(memory_space=pltpu.SMEM),
        out_shape=jax.ShapeDtypeStruct((4,), jnp.float32),
        scratch_shapes=[pltpu.SMEM((16,), jnp.float32)],
    )(cells0, cells1, cells2, tt, img, objs)


def kernel(preds_0, preds_1, preds_2, targets, image_size):
    pt0 = jnp.transpose(preds_0, (0, 2, 3, 1))
    pt1 = jnp.transpose(preds_1, (0, 2, 3, 1))
    pt2 = jnp.transpose(preds_2, (2, 3, 0, 1))
    tt = targets[0].astype(jnp.float32)
    ttc = tt.T
    img = image_size.reshape(1, 2).astype(jnp.float32)
    img32 = jnp.concatenate([jnp.full((16,), image_size[0], jnp.float32),
                             jnp.full((16,), image_size[1], jnp.float32)])
    args = (ttc[0], ttc[2], ttc[3], img32)
    cells0 = _sc_gather_fn(0)(pt0, *args).reshape(_NCELL, 8, 255)
    cells1 = _sc_gather_fn(1)(pt1, *args).reshape(_NCELL, 8, 255)
    cells2 = _sc_gather_fn(2)(pt2, *args).reshape(_NCELL, 8, 255)
    objs = _obj_main(pt0, pt1, pt2)
    out = _loss_main(cells0, cells1, cells2, tt, img, objs)
    return (out[0:1], out[1:2], out[2:3], out[3:4])


# speedup vs baseline: 1.0420x; 1.0420x over previous
import functools

import jax
import jax.numpy as jnp
import numpy as np
from jax import lax
from jax.experimental import pallas as pl
from jax.experimental.pallas import tpu as pltpu
from jax.experimental.pallas import tpu_sc as plsc

_ANCHORS = np.array(
    [10, 13, 16, 30, 33, 23, 30, 61, 62, 45, 59, 119, 116, 90, 156, 198, 373, 326],
    dtype=np.float32,
).reshape(3, 3, 2)
_BAL = (4.0, 1.0, 0.4)
_OFF = ((0.0, 0.0), (1.0, 0.0), (0.0, 1.0), (-1.0, 0.0), (0.0, -1.0))
_GRIDS = ((80, 80), (40, 40), (20, 20))
_B = 16
_N = 256
_NCELL = 5 * _N
_EPS = 1e-7


def _softplus0(x):
    return jnp.maximum(x, 0.0) + jnp.log1p(jnp.exp(-jnp.abs(x)))


_ATAN_C = (9.9999999755e-01, -3.3333282296e-01, 1.9998230640e-01,
           -1.4261573680e-01, 1.0940198965e-01, -8.3720639484e-02,
           5.7463557856e-02, -3.0717508912e-02, 1.0680719451e-02,
           -1.7437011450e-03)


def _atan_pos(z):
    big = z > 1.0
    t = jnp.where(big, 1.0 / z, z)
    u = t * t
    p = jnp.full_like(u, _ATAN_C[-1])
    for c in _ATAN_C[-2::-1]:
        p = p * u + c
    at = t * p
    return jnp.where(big, (np.pi / 2) - at, at)


def _make_sc_body(lvl):
    H, W = _GRIDS[lvl]

    def _sc_body(pt, nb_hbm, cx_hbm, cy_hbm, img_hbm, cells,
                 nbv, cxv, cyv, imgv, idxv, slab, sem):
        nc = 2
        w = lax.axis_index("s") * nc + lax.axis_index("c")
        pltpu.sync_copy(nb_hbm, nbv.at[pl.ds(0, _N)])
        pltpu.sync_copy(nb_hbm, nbv.at[pl.ds(_N, _N)])
        pltpu.sync_copy(cx_hbm, cxv.at[pl.ds(0, _N)])
        pltpu.sync_copy(cx_hbm, cxv.at[pl.ds(_N, _N)])
        pltpu.sync_copy(cy_hbm, cyv.at[pl.ds(0, _N)])
        pltpu.sync_copy(cy_hbm, cyv.at[pl.ds(_N, _N)])
        pltpu.sync_copy(img_hbm, imgv)
        imgw = imgv[pl.ds(0, 16)]
        imgh = imgv[pl.ds(16, 16)]
        lane = lax.iota(jnp.int32, 16)
        base = w * 40
        gx_scale = 1.0 / (imgw / float(W))
        gy_scale = 1.0 / (imgh / float(H))
        for off in (0, 16, 32):
            ids = jnp.minimum(base + off + lane, _NCELL - 1)
            o = lax.shift_right_logical(ids, 8)
            xo = (jnp.where(o == 1, 1.0, 0.0)
                  + jnp.where(o == 3, -1.0, 0.0))
            yo = (jnp.where(o == 2, 1.0, 0.0)
                  + jnp.where(o == 4, -1.0, 0.0))
            t0 = (base + off) % _N
            cxt = cxv[pl.ds(t0, 16)] * gx_scale - xo
            cyt = cyv[pl.ds(t0, 16)] * gy_scale - yo
            nbt = nbv[pl.ds(t0, 16)]
            gi = jnp.clip(cxt.astype(jnp.int32), 0, W - 1)
            gj = jnp.clip(cyt.astype(jnp.int32), 0, H - 1)
            b = jnp.clip(nbt.astype(jnp.int32), 0, _B - 1)
            idxv[pl.ds(off, 16)] = b
            idxv[pl.ds(48 + off, 16)] = gj
            idxv[pl.ds(96 + off, 16)] = gi

        gb = [idxv[pl.ds(0, 16)], idxv[pl.ds(16, 16)], idxv[pl.ds(32, 16)]]
        gjv = [idxv[pl.ds(48, 16)], idxv[pl.ds(64, 16)], idxv[pl.ds(80, 16)]]
        giv = [idxv[pl.ds(96, 16)], idxv[pl.ds(112, 16)],
               idxv[pl.ds(128, 16)]]
        for k in range(40):
            q, r = divmod(k, 16)
            b = gb[q][r]
            gj = gjv[q][r]
            gi = giv[q][r]
            if lvl == 2:
                ba = pl.multiple_of((b // 8) * 8, 8)
                src = pt.at[pl.ds(gj, 1), pl.ds(gi, 1), pl.ds(ba, 8)]
            else:
                gia = pl.multiple_of((gi // 8) * 8, 8)
                src = pt.at[pl.ds(b, 1), pl.ds(gj, 1), pl.ds(gia, 8)]
            pltpu.async_copy(src, slab.at[pl.ds(k, 1), pl.ds(0, 1)], sem)
        pltpu.make_async_copy(cells.at[pl.ds(0, 40)], slab, sem).wait()
        pltpu.sync_copy(slab, cells.at[pl.ds(base, 40)])

    return _sc_body


@functools.cache
def _sc_gather_fn(lvl):
    mesh = plsc.VectorSubcoreMesh(core_axis_name="c", subcore_axis_name="s")
    return pl.kernel(
        _make_sc_body(lvl),
        mesh=mesh,
        out_type=jax.ShapeDtypeStruct((_NCELL, 1, 8, 255), jnp.float32),
        scratch_types=[
            pltpu.VMEM((512,), jnp.float32),
            pltpu.VMEM((512,), jnp.float32),
            pltpu.VMEM((512,), jnp.float32),
            pltpu.VMEM((32,), jnp.float32),
            pltpu.VMEM((144,), jnp.int32),
            pltpu.VMEM((40, 1, 8, 255), jnp.float32),
            pltpu.SemaphoreType.DMA,
        ],
    )


def _sel_matrix():
    row = lax.broadcasted_iota(jnp.int32, (255, 128), 0)
    col = lax.broadcasted_iota(jnp.int32, (255, 128), 1)
    sel = (((row == 4) & (col == 0)) | ((row == 89) & (col == 1))
           | ((row == 174) & (col == 2)))
    return sel.astype(jnp.float32)


def _obj_extract(x2d, sel):
    y = jax.lax.dot_general(x2d, sel, (((1,), (0,)), ((), ())),
                            preferred_element_type=jnp.float32)
    return jnp.sum(_softplus0(y[:, 0:3]))


def _obj_body(p0_ref, p1_ref, p2_ref, out_ref, acc_ref):
    g = pl.program_id(0)
    sel = _sel_matrix()

    @pl.when(g == 0)
    def _init():
        acc_ref[0] = 0.0
        acc_ref[1] = 0.0
        acc_ref[2] = 0.0

    @pl.when(g < 160)
    def _l01():
        acc_ref[0] += _obj_extract(p0_ref[...].reshape(8 * 80, 255), sel)
        acc_ref[1] += _obj_extract(p1_ref[...].reshape(4 * 40, 255), sel)

    @pl.when(g >= 160)
    def _l2():
        acc_ref[2] += _obj_extract(p2_ref[...].reshape(2 * 20 * _B, 255), sel)

    @pl.when(g == pl.num_programs(0) - 1)
    def _final():
        out_ref[0] = acc_ref[0]
        out_ref[1] = acc_ref[1]
        out_ref[2] = acc_ref[2]
        out_ref[3] = 0.0


def _obj_main(pt0, pt1, pt2):
    def m0(g):
        gc = jnp.minimum(g, 159)
        return (gc // 10, gc % 10, 0, 0)

    def m2(g):
        return (jnp.maximum(g, 160) - 160, 0, 0, 0)

    return pl.pallas_call(
        _obj_body,
        grid=(170,),
        in_specs=[
            pl.BlockSpec((1, 8, 80, 255), m0),
            pl.BlockSpec((1, 4, 40, 255), m0),
            pl.BlockSpec((2, 20, _B, 255), m2),
        ],
        out_specs=pl.BlockSpec(memory_space=pltpu.SMEM),
        out_shape=jax.ShapeDtypeStruct((4,), jnp.float32),
        scratch_shapes=[pltpu.SMEM((4,), jnp.float32)],
    )(pt0, pt1, pt2)


def _loss_body(c0_ref, c1_ref, c2_ref, tt_ref, img_ref, objs_ref, out_ref,
               acc_ref):
    g = pl.program_id(0)
    imgw = img_ref[0, 0]
    imgh = img_ref[0, 1]
    cls_ = tt_ref[:, 1:2]
    xo = jnp.where(g == 1, 1.0, 0.0) + jnp.where(g == 3, -1.0, 0.0)
    yo = jnp.where(g == 2, 1.0, 0.0) + jnp.where(g == 4, -1.0, 0.0)

    @pl.when(g == 0)
    def _init():
        for i in range(12):
            acc_ref[3 + i] = 0.0

    for lvl, (H, W) in enumerate(_GRIDS):
        c_ref = (c0_ref, c1_ref, c2_ref)[lvl]
        sx = imgw / W
        sy = imgh / H
        gxs = 1.0 / sx
        gys = 1.0 / sy
        cx = tt_ref[:, 2:3] * gxs
        cy = tt_ref[:, 3:4] * gys
        gw = tt_ref[:, 4:5] * gxs
        gh = tt_ref[:, 5:6] * gys
        cidx = jnp.clip(cls_.astype(jnp.int32) - 1, 0, 79)
        oneh = (lax.broadcasted_iota(jnp.int32, (_N, 80), 1)
                == cidx).astype(jnp.float32)
        gxf = cx - xo
        gyf = cy - yo
        j1 = (gxf >= 0) & (gxf < W) & (gyf >= 0) & (gyf < H)
        gxi = gxf.astype(jnp.int32).astype(jnp.float32)
        gyi = gyf.astype(jnp.int32).astype(jnp.float32)
        if lvl == 2:
            bi = jnp.clip(tt_ref[:, 0:1].astype(jnp.int32), 0, _B - 1)
            rt = bi - 8 * (bi // 8)
        else:
            gii = jnp.clip(gxf.astype(jnp.int32), 0, W - 1)
            rt = gii - 8 * (gii // 8)
        psc255 = c_ref[:, 0, :] * (rt == 0).astype(jnp.float32)
        for r in range(1, 8):
            psc255 += c_ref[:, r, :] * (rt == r).astype(jnp.float32)
        box_s = 0.0
        corr_s = 0.0
        cls_s = 0.0
        nv_s = 0.0
        for a in range(3):
            aw = _ANCHORS[lvl, a, 0] / sx
            ah = _ANCHORS[lvl, a, 1] / sy
            rw = gw / aw
            rh = gh / ah
            j2 = (jnp.maximum(jnp.maximum(rw, 1.0 / rw),
                              jnp.maximum(rh, 1.0 / rh)) < 4.0)
            mf = jnp.where(j1 & j2, 1.0, 0.0)
            psc = psc255[:, 85 * a:85 * a + 85]
            px = 3.0 * jax.nn.sigmoid(psc[:, 0:1]) - 1.0
            py = 3.0 * jax.nn.sigmoid(psc[:, 1:2]) - 1.0
            sw = jax.nn.sigmoid(psc[:, 2:3])
            sh = jax.nn.sigmoid(psc[:, 3:4])
            pw = 4.0 * sw * sw * aw
            ph = 4.0 * sh * sh * ah
            tbx = cx - gxi
            tby = cy - gyi
            b1x1 = px - pw * 0.5
            b1x2 = px + pw * 0.5
            b1y1 = py - ph * 0.5
            b1y2 = py + ph * 0.5
            b2x1 = tbx - gw * 0.5
            b2x2 = tbx + gw * 0.5
            b2y1 = tby - gh * 0.5
            b2y2 = tby + gh * 0.5
            inter = (jnp.clip(jnp.minimum(b1x2, b2x2)
                              - jnp.maximum(b1x1, b2x1), 0.0)
                     * jnp.clip(jnp.minimum(b1y2, b2y2)
                                - jnp.maximum(b1y1, b2y1), 0.0))
            union = pw * ph + gw * gh - inter + _EPS
            iou = inter / union
            cw = jnp.maximum(b1x2, b2x2) - jnp.minimum(b1x1, b2x1)
            chh = jnp.maximum(b1y2, b2y2) - jnp.minimum(b1y1, b2y1)
            c2 = cw * cw + chh * chh + _EPS
            rho2 = ((b2x1 + b2x2 - b1x1 - b1x2) ** 2
                    + (b2y1 + b2y2 - b1y1 - b1y2) ** 2) * 0.25
            v = ((4.0 / np.pi ** 2)
                 * (_atan_pos(gw / (gh + _EPS))
                    - _atan_pos(pw / (ph + _EPS))) ** 2)
            alpha = v / (v - iou + (1.0 + _EPS))
            ciou = iou - (rho2 / c2 + v * alpha)
            box_s += jnp.sum((1.0 - ciou) * mf)
            corr_s += jnp.sum(psc[:, 4:5] * jnp.clip(ciou, 0.0) * mf)
            nv_s += jnp.sum(mf)
            xl = psc[:, 5:85]
            bce = (jnp.maximum(xl, 0.0) - xl * oneh
                   + jnp.log1p(jnp.exp(-jnp.abs(xl))))
            cls_s += jnp.sum(bce * mf)
        acc_ref[3 + lvl] += box_s
        acc_ref[6 + lvl] += corr_s
        acc_ref[9 + lvl] += cls_s
        acc_ref[12 + lvl] += nv_s

    @pl.when(g == pl.num_programs(0) - 1)
    def _final():
        lbox = 0.0
        lobj = 0.0
        lcls = 0.0
        for lvl, (H, W) in enumerate(_GRIDS):
            denom = jnp.maximum(acc_ref[12 + lvl], 1.0)
            lbox += acc_ref[3 + lvl] / denom
            lcls += acc_ref[9 + lvl] / (denom * 80.0)
            lobj += ((objs_ref[lvl] - acc_ref[6 + lvl])
                     / (_B * 3 * H * W)) * _BAL[lvl]
        lbox = lbox * 3.54
        lobj = lobj * 64.3
        lcls = lcls * 37.4
        loss = lbox + lobj + lcls
        out_ref[0] = loss
        out_ref[1] = lbox
        out_ref[2] = lobj
        out_ref[3] = lcls


def _loss_main(cells0, cells1, cells2, tt, img, objs):
    return pl.pallas_call(
        _loss_body,
        grid=(5,),
        in_specs=[
            pl.BlockSpec((_N, 8, 255), lambda g: (g, 0, 0)),
            pl.BlockSpec((_N, 8, 255), lambda g: (g, 0, 0)),
            pl.BlockSpec((_N, 8, 255), lambda g: (g, 0, 0)),
            pl.BlockSpec((_N, 6), lambda g: (0, 0)),
            pl.BlockSpec(memory_space=pltpu.SMEM),
            pl.BlockSpec(memory_space=pltpu.SMEM),
        ],
        out_specs=pl.BlockSpec(memory_space=pltpu.SMEM),
        out_shape=jax.ShapeDtypeStruct((4,), jnp.float32),
        scratch_shapes=[pltpu.SMEM((16,), jnp.float32)],
    )(cells0, cells1, cells2, tt, img, objs)


def kernel(preds_0, preds_1, preds_2, targets, image_size):
    pt0 = jnp.transpose(preds_0, (0, 2, 3, 1))
    pt1 = jnp.transpose(preds_1, (0, 2, 3, 1))
    pt2 = jnp.transpose(preds_2, (2, 3, 0, 1))
    tt = targets[0].astype(jnp.float32)
    ttc = tt.T
    img = image_size.reshape(1, 2).astype(jnp.float32)
    img32 = jnp.concatenate([jnp.full((16,), image_size[0], jnp.float32),
                             jnp.full((16,), image_size[1], jnp.float32)])
    args = (ttc[0], ttc[2], ttc[3], img32)
    cells0 = _sc_gather_fn(0)(pt0, *args).reshape(_NCELL, 8, 255)
    cells1 = _sc_gather_fn(1)(pt1, *args).reshape(_NCELL, 8, 255)
    cells2 = _sc_gather_fn(2)(pt2, *args).reshape(_NCELL, 8, 255)
    objs = _obj_main(pt0, pt1, pt2)
    out = _loss_main(cells0, cells1, cells2, tt, img, objs)
    return (out[0:1], out[1:2], out[2:3], out[3:4])


# speedup vs baseline: 2.0323x; 1.9504x over previous
import functools

import jax
import jax.numpy as jnp
import numpy as np
from jax import lax
from jax.experimental import pallas as pl
from jax.experimental.pallas import tpu as pltpu
from jax.experimental.pallas import tpu_sc as plsc

_ANCHORS = np.array(
    [10, 13, 16, 30, 33, 23, 30, 61, 62, 45, 59, 119, 116, 90, 156, 198, 373, 326],
    dtype=np.float32,
).reshape(3, 3, 2)
_BAL = (4.0, 1.0, 0.4)
_OFF = ((0.0, 0.0), (1.0, 0.0), (0.0, 1.0), (-1.0, 0.0), (0.0, -1.0))
_GRIDS = ((80, 80), (40, 40), (20, 20))
_B = 16
_N = 256
_NCELL = 5 * _N
_EPS = 1e-7


def _softplus0(x):
    return jnp.maximum(x, 0.0) + jnp.log1p(jnp.exp(-jnp.abs(x)))


_ATAN_C = (9.9999999755e-01, -3.3333282296e-01, 1.9998230640e-01,
           -1.4261573680e-01, 1.0940198965e-01, -8.3720639484e-02,
           5.7463557856e-02, -3.0717508912e-02, 1.0680719451e-02,
           -1.7437011450e-03)


def _atan_pos(z):
    big = z > 1.0
    t = jnp.where(big, 1.0 / z, z)
    u = t * t
    p = jnp.full_like(u, _ATAN_C[-1])
    for c in _ATAN_C[-2::-1]:
        p = p * u + c
    at = t * p
    return jnp.where(big, (np.pi / 2) - at, at)


def _make_sc_body(lvl):
    H, W = _GRIDS[lvl]

    def _sc_body(pt, nb_hbm, cx_hbm, cy_hbm, img_hbm, cells,
                 nbv, cxv, cyv, imgv, idxv, slab, sem):
        nc = 2
        w = lax.axis_index("s") * nc + lax.axis_index("c")
        pltpu.sync_copy(nb_hbm, nbv.at[pl.ds(0, _N)])
        pltpu.sync_copy(nb_hbm, nbv.at[pl.ds(_N, _N)])
        pltpu.sync_copy(cx_hbm, cxv.at[pl.ds(0, _N)])
        pltpu.sync_copy(cx_hbm, cxv.at[pl.ds(_N, _N)])
        pltpu.sync_copy(cy_hbm, cyv.at[pl.ds(0, _N)])
        pltpu.sync_copy(cy_hbm, cyv.at[pl.ds(_N, _N)])
        pltpu.sync_copy(img_hbm, imgv)
        imgw = imgv[pl.ds(0, 16)]
        imgh = imgv[pl.ds(16, 16)]
        lane = lax.iota(jnp.int32, 16)
        base = w * 40
        gx_scale = 1.0 / (imgw / float(W))
        gy_scale = 1.0 / (imgh / float(H))
        for off in (0, 16, 32):
            ids = jnp.minimum(base + off + lane, _NCELL - 1)
            o = lax.shift_right_logical(ids, 8)
            xo = (jnp.where(o == 1, 1.0, 0.0)
                  + jnp.where(o == 3, -1.0, 0.0))
            yo = (jnp.where(o == 2, 1.0, 0.0)
                  + jnp.where(o == 4, -1.0, 0.0))
            t0 = (base + off) % _N
            cxt = cxv[pl.ds(t0, 16)] * gx_scale - xo
            cyt = cyv[pl.ds(t0, 16)] * gy_scale - yo
            nbt = nbv[pl.ds(t0, 16)]
            gi = jnp.clip(cxt.astype(jnp.int32), 0, W - 1)
            gj = jnp.clip(cyt.astype(jnp.int32), 0, H - 1)
            b = jnp.clip(nbt.astype(jnp.int32), 0, _B - 1)
            idxv[pl.ds(off, 16)] = b
            idxv[pl.ds(48 + off, 16)] = gj
            idxv[pl.ds(96 + off, 16)] = gi

        gb = [idxv[pl.ds(0, 16)], idxv[pl.ds(16, 16)], idxv[pl.ds(32, 16)]]
        gjv = [idxv[pl.ds(48, 16)], idxv[pl.ds(64, 16)], idxv[pl.ds(80, 16)]]
        giv = [idxv[pl.ds(96, 16)], idxv[pl.ds(112, 16)],
               idxv[pl.ds(128, 16)]]
        for k in range(40):
            q, r = divmod(k, 16)
            b = gb[q][r]
            gj = gjv[q][r]
            gi = giv[q][r]
            if lvl == 2:
                ba = pl.multiple_of((b // 8) * 8, 8)
                src = pt.at[pl.ds(gj, 1), pl.ds(gi, 1), pl.ds(ba, 8)]
            else:
                gia = pl.multiple_of((gi // 8) * 8, 8)
                src = pt.at[pl.ds(b, 1), pl.ds(gj, 1), pl.ds(gia, 8)]
            pltpu.async_copy(src, slab.at[pl.ds(k, 1), pl.ds(0, 1)], sem)
        pltpu.make_async_copy(cells.at[pl.ds(0, 40)], slab, sem).wait()
        pltpu.sync_copy(slab, cells.at[pl.ds(base, 40)])

    return _sc_body


@functools.cache
def _sc_gather_fn(lvl):
    mesh = plsc.VectorSubcoreMesh(core_axis_name="c", subcore_axis_name="s")
    return pl.kernel(
        _make_sc_body(lvl),
        mesh=mesh,
        out_type=jax.ShapeDtypeStruct((_NCELL, 1, 8, 255), jnp.float32),
        scratch_types=[
            pltpu.VMEM((512,), jnp.float32),
            pltpu.VMEM((512,), jnp.float32),
            pltpu.VMEM((512,), jnp.float32),
            pltpu.VMEM((32,), jnp.float32),
            pltpu.VMEM((144,), jnp.int32),
            pltpu.VMEM((40, 1, 8, 255), jnp.float32),
            pltpu.SemaphoreType.DMA,
        ],
    )


def _sel_matrix():
    row = lax.broadcasted_iota(jnp.int32, (255, 128), 0)
    col = lax.broadcasted_iota(jnp.int32, (255, 128), 1)
    sel = (((row == 4) & (col == 0)) | ((row == 89) & (col == 1))
           | ((row == 174) & (col == 2)))
    return sel.astype(jnp.float32)


def _obj_extract(x2d, sel):
    y = jax.lax.dot_general(x2d, sel, (((1,), (0,)), ((), ())),
                            preferred_element_type=jnp.float32)
    return jnp.sum(_softplus0(y[:, 0:3]))


def _obj_body(p0_ref, p1_ref, p2_ref, out_ref, acc_ref):
    g = pl.program_id(0)
    sel = _sel_matrix()

    @pl.when(g == 0)
    def _init():
        acc_ref[0] = 0.0
        acc_ref[1] = 0.0
        acc_ref[2] = 0.0

    @pl.when(g < 160)
    def _l01():
        acc_ref[0] += _obj_extract(p0_ref[...].reshape(8 * 80, 255), sel)
        acc_ref[1] += _obj_extract(p1_ref[...].reshape(4 * 40, 255), sel)

    @pl.when(g >= 160)
    def _l2():
        acc_ref[2] += _obj_extract(p2_ref[...].reshape(2 * 20 * _B, 255), sel)

    @pl.when(g == pl.num_programs(0) - 1)
    def _final():
        out_ref[0] = acc_ref[0]
        out_ref[1] = acc_ref[1]
        out_ref[2] = acc_ref[2]
        out_ref[3] = 0.0


def _obj_main(pt0, pt1, pt2):
    def m0(g):
        gc = jnp.minimum(g, 159)
        return (gc // 10, gc % 10, 0, 0)

    def m2(g):
        return (jnp.maximum(g, 160) - 160, 0, 0, 0)

    return pl.pallas_call(
        _obj_body,
        grid=(170,),
        in_specs=[
            pl.BlockSpec((1, 8, 80, 255), m0),
            pl.BlockSpec((1, 4, 40, 255), m0),
            pl.BlockSpec((2, 20, _B, 255), m2),
        ],
        out_specs=pl.BlockSpec(memory_space=pltpu.SMEM),
        out_shape=jax.ShapeDtypeStruct((4,), jnp.float32),
        scratch_shapes=[pltpu.SMEM((4,), jnp.float32)],
    )(pt0, pt1, pt2)


def _loss_body(c0_ref, c1_ref, c2_ref, tt_ref, img_ref, objs_ref, out_ref,
               acc_ref):
    g = pl.program_id(0)
    imgw = img_ref[0, 0]
    imgh = img_ref[0, 1]
    cls_ = tt_ref[:, 1:2]
    xo = jnp.where(g == 1, 1.0, 0.0) + jnp.where(g == 3, -1.0, 0.0)
    yo = jnp.where(g == 2, 1.0, 0.0) + jnp.where(g == 4, -1.0, 0.0)

    @pl.when(g == 0)
    def _init():
        for i in range(12):
            acc_ref[3 + i] = 0.0

    for lvl, (H, W) in enumerate(_GRIDS):
        c_ref = (c0_ref, c1_ref, c2_ref)[lvl]
        sx = imgw / W
        sy = imgh / H
        gxs = 1.0 / sx
        gys = 1.0 / sy
        cx = tt_ref[:, 2:3] * gxs
        cy = tt_ref[:, 3:4] * gys
        gw = tt_ref[:, 4:5] * gxs
        gh = tt_ref[:, 5:6] * gys
        cidx = jnp.clip(cls_.astype(jnp.int32) - 1, 0, 79)
        gxf = cx - xo
        gyf = cy - yo
        j1 = (gxf >= 0) & (gxf < W) & (gyf >= 0) & (gyf < H)
        gxi = gxf.astype(jnp.int32).astype(jnp.float32)
        gyi = gyf.astype(jnp.int32).astype(jnp.float32)
        if lvl == 2:
            bi = jnp.clip(tt_ref[:, 0:1].astype(jnp.int32), 0, _B - 1)
            rt = bi - 8 * (bi // 8)
        else:
            gii = jnp.clip(gxf.astype(jnp.int32), 0, W - 1)
            rt = gii - 8 * (gii // 8)
        rsel = (lax.broadcasted_iota(jnp.int32, (_N, 8, 255), 1)
                == rt[:, :, None]).astype(jnp.float32)
        psc255 = jnp.sum(c_ref[...] * rsel, axis=1)
        brow = lax.broadcasted_iota(jnp.int32, (255, 16), 0)
        bcol = lax.broadcasted_iota(jnp.int32, (255, 16), 1)
        bsel = ((brow == (bcol % 5) + 85 * (bcol // 5)) & (bcol < 15))
        bb = jax.lax.dot_general(psc255, bsel.astype(jnp.float32),
                                 (((1,), (0,)), ((), ())),
                                 preferred_element_type=jnp.float32)
        lanes = lax.broadcasted_iota(jnp.int32, (_N, 255), 1)
        oneh255 = ((lanes == cidx + 5) | (lanes == cidx + 90)
                   | (lanes == cidx + 175)).astype(jnp.float32)
        bce_all = (jnp.maximum(psc255, 0.0) - psc255 * oneh255
                   + jnp.log1p(jnp.exp(-jnp.abs(psc255))))
        box_s = 0.0
        corr_s = 0.0
        cls_s = 0.0
        nv_s = 0.0
        for a in range(3):
            aw = _ANCHORS[lvl, a, 0] / sx
            ah = _ANCHORS[lvl, a, 1] / sy
            rw = gw / aw
            rh = gh / ah
            j2 = (jnp.maximum(jnp.maximum(rw, 1.0 / rw),
                              jnp.maximum(rh, 1.0 / rh)) < 4.0)
            mf = jnp.where(j1 & j2, 1.0, 0.0)
            px = 3.0 * jax.nn.sigmoid(bb[:, 5 * a:5 * a + 1]) - 1.0
            py = 3.0 * jax.nn.sigmoid(bb[:, 5 * a + 1:5 * a + 2]) - 1.0
            sw = jax.nn.sigmoid(bb[:, 5 * a + 2:5 * a + 3])
            sh = jax.nn.sigmoid(bb[:, 5 * a + 3:5 * a + 4])
            pw = 4.0 * sw * sw * aw
            ph = 4.0 * sh * sh * ah
            tbx = cx - gxi
            tby = cy - gyi
            b1x1 = px - pw * 0.5
            b1x2 = px + pw * 0.5
            b1y1 = py - ph * 0.5
            b1y2 = py + ph * 0.5
            b2x1 = tbx - gw * 0.5
            b2x2 = tbx + gw * 0.5
            b2y1 = tby - gh * 0.5
            b2y2 = tby + gh * 0.5
            inter = (jnp.clip(jnp.minimum(b1x2, b2x2)
                              - jnp.maximum(b1x1, b2x1), 0.0)
                     * jnp.clip(jnp.minimum(b1y2, b2y2)
                                - jnp.maximum(b1y1, b2y1), 0.0))
            union = pw * ph + gw * gh - inter + _EPS
            iou = inter / union
            cw = jnp.maximum(b1x2, b2x2) - jnp.minimum(b1x1, b2x1)
            chh = jnp.maximum(b1y2, b2y2) - jnp.minimum(b1y1, b2y1)
            c2 = cw * cw + chh * chh + _EPS
            rho2 = ((b2x1 + b2x2 - b1x1 - b1x2) ** 2
                    + (b2y1 + b2y2 - b1y1 - b1y2) ** 2) * 0.25
            v = ((4.0 / np.pi ** 2)
                 * (_atan_pos(gw / (gh + _EPS))
                    - _atan_pos(pw / (ph + _EPS))) ** 2)
            alpha = v / (v - iou + (1.0 + _EPS))
            ciou = iou - (rho2 / c2 + v * alpha)
            box_s += jnp.sum((1.0 - ciou) * mf)
            corr_s += jnp.sum(bb[:, 5 * a + 4:5 * a + 5]
                              * jnp.clip(ciou, 0.0) * mf)
            nv_s += jnp.sum(mf)
            clsmask = ((lanes >= 85 * a + 5)
                       & (lanes < 85 * a + 85)).astype(jnp.float32)
            cls_s += jnp.sum(bce_all * clsmask * mf)
        acc_ref[3 + lvl] += box_s
        acc_ref[6 + lvl] += corr_s
        acc_ref[9 + lvl] += cls_s
        acc_ref[12 + lvl] += nv_s

    @pl.when(g == pl.num_programs(0) - 1)
    def _final():
        lbox = 0.0
        lobj = 0.0
        lcls = 0.0
        for lvl, (H, W) in enumerate(_GRIDS):
            denom = jnp.maximum(acc_ref[12 + lvl], 1.0)
            lbox += acc_ref[3 + lvl] / denom
            lcls += acc_ref[9 + lvl] / (denom * 80.0)
            lobj += ((objs_ref[lvl] - acc_ref[6 + lvl])
                     / (_B * 3 * H * W)) * _BAL[lvl]
        lbox = lbox * 3.54
        lobj = lobj * 64.3
        lcls = lcls * 37.4
        loss = lbox + lobj + lcls
        out_ref[0] = loss
        out_ref[1] = lbox
        out_ref[2] = lobj
        out_ref[3] = lcls


def _loss_main(cells0, cells1, cells2, tt, img, objs):
    return pl.pallas_call(
        _loss_body,
        grid=(5,),
        in_specs=[
            pl.BlockSpec((_N, 8, 255), lambda g: (g, 0, 0)),
            pl.BlockSpec((_N, 8, 255), lambda g: (g, 0, 0)),
            pl.BlockSpec((_N, 8, 255), lambda g: (g, 0, 0)),
            pl.BlockSpec((_N, 6), lambda g: (0, 0)),
            pl.BlockSpec(memory_space=pltpu.SMEM),
            pl.BlockSpec(memory_space=pltpu.SMEM),
        ],
        out_specs=pl.BlockSpec(memory_space=pltpu.SMEM),
        out_shape=jax.ShapeDtypeStruct((4,), jnp.float32),
        scratch_shapes=[pltpu.SMEM((16,), jnp.float32)],
    )(cells0, cells1, cells2, tt, img, objs)


def kernel(preds_0, preds_1, preds_2, targets, image_size):
    pt0 = jnp.transpose(preds_0, (0, 2, 3, 1))
    pt1 = jnp.transpose(preds_1, (0, 2, 3, 1))
    pt2 = jnp.transpose(preds_2, (2, 3, 0, 1))
    tt = targets[0].astype(jnp.float32)
    ttc = tt.T
    img = image_size.reshape(1, 2).astype(jnp.float32)
    img32 = jnp.concatenate([jnp.full((16,), image_size[0], jnp.float32),
                             jnp.full((16,), image_size[1], jnp.float32)])
    args = (ttc[0], ttc[2], ttc[3], img32)
    cells0 = _sc_gather_fn(0)(pt0, *args).reshape(_NCELL, 8, 255)
    cells1 = _sc_gather_fn(1)(pt1, *args).reshape(_NCELL, 8, 255)
    cells2 = _sc_gather_fn(2)(pt2, *args).reshape(_NCELL, 8, 255)
    objs = _obj_main(pt0, pt1, pt2)
    out = _loss_main(cells0, cells1, cells2, tt, img, objs)
    return (out[0:1], out[1:2], out[2:3], out[3:4])
